# baseline (speedup 1.0000x reference)
import jax
import jax.numpy as jnp
from jax.experimental import pallas as pl

_BS, _NR, _PLAN = 1024, 26, 20
_VOCAB, _POS, _DIM = 18, 20, 128
_BB = 256
_GRID = _BS // _BB
_C = _BB * _NR

_DN0 = (((0,), (0,)), ((), ()))


def _body(fp_ref, mask_ref, tt_ref, pt_ref, wt_ref, b_ref, out_ref):
    fpt = fp_ref[...].reshape(_C, _PLAN).T
    keept = 1.0 - mask_ref[...].reshape(_C, _PLAN).T.astype(jnp.float32)

    tok_w = jnp.dot(tt_ref[...], wt_ref[...], preferred_element_type=jnp.float32)
    pos_wb = jnp.dot(pt_ref[...], wt_ref[...], preferred_element_type=jnp.float32) + b_ref[...]

    cols = [
        jnp.sum(jnp.where(fpt == v, keept, 0.0), axis=0, keepdims=True)
        for v in range(_VOCAB)
    ]
    counts_t = jnp.concatenate(cols, axis=0)

    out = (
        jax.lax.dot_general(counts_t, tok_w, _DN0, preferred_element_type=jnp.float32)
        + jax.lax.dot_general(keept, pos_wb, _DN0, preferred_element_type=jnp.float32)
    )
    out_ref[...] = out.reshape(_BB, _NR, _DIM)


def kernel(fleet_plan, fleet_plan_mask, token_table, pos_table, fc_w, fc_b):
    out = pl.pallas_call(
        _body,
        grid=(_GRID,),
        in_specs=[
            pl.BlockSpec((_BB, _NR, _PLAN), lambda i: (i, 0, 0)),
            pl.BlockSpec((_BB, _NR, _PLAN), lambda i: (i, 0, 0)),
            pl.BlockSpec((_VOCAB, _DIM), lambda i: (0, 0)),
            pl.BlockSpec((_POS, _DIM), lambda i: (0, 0)),
            pl.BlockSpec((_DIM, _DIM), lambda i: (0, 0)),
            pl.BlockSpec((1, _DIM), lambda i: (0, 0)),
        ],
        out_specs=pl.BlockSpec((_BB, _NR, _DIM), lambda i: (i, 0, 0)),
        out_shape=jax.ShapeDtypeStruct((_BS, _NR, _DIM), jnp.float32),
    )(fleet_plan.astype(jnp.int32), fleet_plan_mask, token_table, pos_table,
      fc_w.T, fc_b.reshape(1, _DIM))
    return out


# speedup vs baseline: 1.0359x; 1.0359x over previous
import jax
import jax.numpy as jnp
from jax.experimental import pallas as pl

_BS, _NR, _PLAN = 1024, 26, 20
_VOCAB, _POS, _DIM = 18, 20, 128
_BB = 64
_GRID = _BS // _BB
_C = _BB * _NR

_DN0 = (((0,), (0,)), ((), ()))


def _body(fp_ref, mask_ref, tt_ref, pt_ref, wt_ref, b_ref, out_ref):
    fpt = fp_ref[...].reshape(_C, _PLAN).T
    keept = 1.0 - mask_ref[...].reshape(_C, _PLAN).T.astype(jnp.float32)

    tok_w = jnp.dot(tt_ref[...], wt_ref[...], preferred_element_type=jnp.float32)
    pos_wb = jnp.dot(pt_ref[...], wt_ref[...], preferred_element_type=jnp.float32) + b_ref[...]

    cols = [
        jnp.sum(jnp.where(fpt == v, keept, 0.0), axis=0, keepdims=True)
        for v in range(_VOCAB)
    ]
    counts_t = jnp.concatenate(cols, axis=0)

    out = (
        jax.lax.dot_general(counts_t, tok_w, _DN0, preferred_element_type=jnp.float32)
        + jax.lax.dot_general(keept, pos_wb, _DN0, preferred_element_type=jnp.float32)
    )
    out_ref[...] = out.reshape(_BB, _NR, _DIM)


def kernel(fleet_plan, fleet_plan_mask, token_table, pos_table, fc_w, fc_b):
    out = pl.pallas_call(
        _body,
        grid=(_GRID,),
        in_specs=[
            pl.BlockSpec((_BB, _NR, _PLAN), lambda i: (i, 0, 0)),
            pl.BlockSpec((_BB, _NR, _PLAN), lambda i: (i, 0, 0)),
            pl.BlockSpec((_VOCAB, _DIM), lambda i: (0, 0)),
            pl.BlockSpec((_POS, _DIM), lambda i: (0, 0)),
            pl.BlockSpec((_DIM, _DIM), lambda i: (0, 0)),
            pl.BlockSpec((1, _DIM), lambda i: (0, 0)),
        ],
        out_specs=pl.BlockSpec((_BB, _NR, _DIM), lambda i: (i, 0, 0)),
        out_shape=jax.ShapeDtypeStruct((_BS, _NR, _DIM), jnp.float32),
    )(fleet_plan.astype(jnp.int32), fleet_plan_mask, token_table, pos_table,
      fc_w.T, fc_b.reshape(1, _DIM))
    return out


# speedup vs baseline: 1.1285x; 1.0894x over previous
import jax
import jax.numpy as jnp
from jax.experimental import pallas as pl
from jax.experimental.pallas import tpu as pltpu

_BS, _NR, _PLAN = 1024, 26, 20
_VOCAB, _POS, _DIM = 18, 20, 128
_N = _BS * _NR
_BB = 256
_GRID = _BS // _BB
_C = _BB * _NR

_DN0 = (((0,), (0,)), ((), ()))


def _body(fpt_ref, maskt_ref, tt_ref, pt_ref, wt_ref, b_ref, out_ref):
    fpt = fpt_ref[...]
    keept = 1.0 - maskt_ref[...].astype(jnp.float32)

    tok_w = jnp.dot(tt_ref[...], wt_ref[...], preferred_element_type=jnp.float32)
    pos_wb = jnp.dot(pt_ref[...], wt_ref[...], preferred_element_type=jnp.float32) + b_ref[...]

    cols = [
        jnp.sum(jnp.where(fpt == v, keept, 0.0), axis=0, keepdims=True)
        for v in range(_VOCAB)
    ]
    counts_t = jnp.concatenate(cols, axis=0)

    out = (
        jax.lax.dot_general(counts_t, tok_w, _DN0, preferred_element_type=jnp.float32)
        + jax.lax.dot_general(keept, pos_wb, _DN0, preferred_element_type=jnp.float32)
    )
    out_ref[...] = out.reshape(_BB, _NR, _DIM)


def kernel(fleet_plan, fleet_plan_mask, token_table, pos_table, fc_w, fc_b):
    fpt = fleet_plan.astype(jnp.int32).transpose(2, 0, 1).reshape(_PLAN, _N)
    maskt = fleet_plan_mask.transpose(2, 0, 1).reshape(_PLAN, _N)

    out = pl.pallas_call(
        _body,
        grid=(_GRID,),
        in_specs=[
            pl.BlockSpec((_PLAN, _C), lambda i: (0, i)),
            pl.BlockSpec((_PLAN, _C), lambda i: (0, i)),
            pl.BlockSpec((_VOCAB, _DIM), lambda i: (0, 0)),
            pl.BlockSpec((_POS, _DIM), lambda i: (0, 0)),
            pl.BlockSpec((_DIM, _DIM), lambda i: (0, 0)),
            pl.BlockSpec((1, _DIM), lambda i: (0, 0)),
        ],
        out_specs=pl.BlockSpec((_BB, _NR, _DIM), lambda i: (i, 0, 0)),
        out_shape=jax.ShapeDtypeStruct((_BS, _NR, _DIM), jnp.float32),
    )(fpt, maskt, token_table, pos_table, fc_w.T, fc_b.reshape(1, _DIM))

    return out


# speedup vs baseline: 1.1452x; 1.0148x over previous
import jax
import jax.numpy as jnp
from jax.experimental import pallas as pl
from jax.experimental.pallas import tpu as pltpu

_BS, _NR, _PLAN = 1024, 26, 20
_VOCAB, _POS, _DIM = 18, 20, 128
_N = _BS * _NR
_BB = 128
_GRID = _BS // _BB
_C = _BB * _NR

_DN0 = (((0,), (0,)), ((), ()))


def _body(fpt_ref, maskt_ref, tt_ref, pt_ref, wt_ref, b_ref, out_ref):
    fpt = fpt_ref[...]
    keept = 1.0 - maskt_ref[...].astype(jnp.float32)

    tok_w = jnp.dot(tt_ref[...], wt_ref[...], preferred_element_type=jnp.float32)
    pos_wb = jnp.dot(pt_ref[...], wt_ref[...], preferred_element_type=jnp.float32) + b_ref[...]

    cols = [
        jnp.sum(jnp.where(fpt == v, keept, 0.0), axis=0, keepdims=True)
        for v in range(_VOCAB)
    ]
    counts_t = jnp.concatenate(cols, axis=0)

    out = (
        jax.lax.dot_general(counts_t, tok_w, _DN0, preferred_element_type=jnp.float32)
        + jax.lax.dot_general(keept, pos_wb, _DN0, preferred_element_type=jnp.float32)
    )
    out_ref[...] = out.reshape(_BB, _NR, _DIM)


def kernel(fleet_plan, fleet_plan_mask, token_table, pos_table, fc_w, fc_b):
    fpt = fleet_plan.astype(jnp.int32).transpose(2, 0, 1).reshape(_PLAN, _N)
    maskt = fleet_plan_mask.transpose(2, 0, 1).reshape(_PLAN, _N)

    out = pl.pallas_call(
        _body,
        grid=(_GRID,),
        in_specs=[
            pl.BlockSpec((_PLAN, _C), lambda i: (0, i)),
            pl.BlockSpec((_PLAN, _C), lambda i: (0, i)),
            pl.BlockSpec((_VOCAB, _DIM), lambda i: (0, 0)),
            pl.BlockSpec((_POS, _DIM), lambda i: (0, 0)),
            pl.BlockSpec((_DIM, _DIM), lambda i: (0, 0)),
            pl.BlockSpec((1, _DIM), lambda i: (0, 0)),
        ],
        out_specs=pl.BlockSpec((_BB, _NR, _DIM), lambda i: (i, 0, 0)),
        out_shape=jax.ShapeDtypeStruct((_BS, _NR, _DIM), jnp.float32),
    )(fpt, maskt, token_table, pos_table, fc_w.T, fc_b.reshape(1, _DIM))

    return out
